# baseline (speedup 1.0000x reference)
import functools

import jax
import jax.numpy as jnp
from jax import lax
from jax.experimental import pallas as pl
from jax.experimental.pallas import tpu as pltpu
from jax.experimental.pallas import tpu_sc as plsc

N_NODES = 10000
N_GRAPHS = 64
D = 128
DQ = 32

NC, NS = 2, 16
CHUNK = 128
N_CHUNKS = 160
NBUF = 4
E_PAD = NS * N_CHUNKS * CHUNK
N_PAD = 10240
ROWS_PER_TILE = N_PAD // NS
DST_PAD = N_NODES + 16

BLK = 2000


def _agg_body(table, srcs, dsts, zeros, out, idx_s, idx_d, rows, acc,
              gsem, ssem):
    cid = lax.axis_index("c")
    sid = lax.axis_index("s")
    pltpu.sync_copy(srcs.at[sid], idx_s)
    pltpu.sync_copy(dsts.at[sid], idx_d)
    r0 = sid * ROWS_PER_TILE
    for q in range(2):
        pltpu.sync_copy(zeros.at[pl.ds(r0, ROWS_PER_TILE)],
                        acc.at[q, pl.ds(r0, ROWS_PER_TILE)])
    plsc.subcore_barrier()
    qs = [table.at[2 * cid], table.at[2 * cid + 1]]
    accq = [acc.at[0], acc.at[1]]
    for b in range(NBUF):
        for q in range(2):
            pltpu.async_copy(qs[q].at[idx_s.at[b]], rows.at[b, q], gsem.at[b])

    def group(g, carry):
        j0 = g * NBUF
        for b in range(NBUF):
            j = j0 + b
            for q in range(2):
                pltpu.make_async_copy(qs[q].at[idx_s.at[j]], rows.at[b, q],
                                      gsem.at[b]).wait()
            for q in range(2):
                pltpu.async_copy(rows.at[b, q], accq[q].at[idx_d.at[j]],
                                 ssem.at[b], add=True)
            pb = (b - 1) % NBUF
            k = j - 1

            @pl.when(k >= 0)
            def _():
                for q in range(2):
                    pltpu.make_async_copy(rows.at[pb, q],
                                          accq[q].at[idx_d.at[k]],
                                          ssem.at[pb]).wait()

                @pl.when(k + NBUF < N_CHUNKS)
                def _():
                    for q in range(2):
                        pltpu.async_copy(qs[q].at[idx_s.at[k + NBUF]],
                                         rows.at[pb, q], gsem.at[pb])
        return carry

    lax.fori_loop(0, N_CHUNKS // NBUF, group, 0)
    for q in range(2):
        pltpu.make_async_copy(rows.at[NBUF - 1, q],
                              accq[q].at[idx_d.at[N_CHUNKS - 1]],
                              ssem.at[NBUF - 1]).wait()
    plsc.subcore_barrier()
    for q in range(2):
        pltpu.sync_copy(acc.at[q, pl.ds(r0, ROWS_PER_TILE)],
                        out.at[2 * cid + q, pl.ds(r0, ROWS_PER_TILE)])


@functools.cache
def _make_agg():
    return pl.kernel(
        _agg_body,
        out_type=jax.ShapeDtypeStruct((4, N_PAD, DQ), jnp.float32),
        mesh=plsc.VectorSubcoreMesh(core_axis_name="c", subcore_axis_name="s"),
        scratch_types=[
            pltpu.VMEM((N_CHUNKS, CHUNK), jnp.int32),
            pltpu.VMEM((N_CHUNKS, CHUNK), jnp.int32),
            pltpu.VMEM((NBUF, 2, CHUNK, DQ), jnp.float32),
            pltpu.VMEM_SHARED((2, N_PAD, DQ), jnp.float32),
            pltpu.SemaphoreType.DMA((NBUF,)),
            pltpu.SemaphoreType.DMA((NBUF,)),
        ],
        compiler_params=pltpu.CompilerParams(use_tc_tiling_on_sc=False),
    )


def _agg(table, srcs, dsts, zeros):
    return _make_agg()(table, srcs, dsts, zeros)


def _split_quarters(mm, o_ref):
    for q in range(4):
        o_ref[q, :, :] = mm[:, DQ * q:DQ * (q + 1)]


def _mm_ones_body(x_ref, w_ref, o_ref):
    xb = x_ref[...]
    x0 = xb[:, :1]
    sq = jnp.sum(xb * xb, axis=1, keepdims=True) - x0 * x0
    ynorm = jnp.clip(jnp.sqrt(jnp.clip(sq, 0.0, None)), 1e-15, None)
    theta = jnp.clip(x0, 1.0 + 1e-7, None)
    ar = jnp.log(theta + jnp.sqrt(jnp.clip(theta * theta - 1.0, 1e-15, None)))
    mm = jnp.dot(xb * (ar / ynorm), w_ref[...],
                 preferred_element_type=jnp.float32)
    col = lax.broadcasted_iota(jnp.int32, mm.shape, 1)
    _split_quarters(jnp.where(col == 0, 1.0, mm), o_ref)


def _mm_ones(xp, wt):
    return pl.pallas_call(
        _mm_ones_body,
        grid=(N_NODES // BLK,),
        in_specs=[pl.BlockSpec((BLK, D), lambda i: (i, 0)),
                  pl.BlockSpec((D, D), lambda i: (0, 0))],
        out_specs=pl.BlockSpec((4, BLK, DQ), lambda i: (0, i, 0)),
        out_shape=jax.ShapeDtypeStruct((4, N_NODES, DQ), jnp.float32),
    )(xp, wt)


def _z_from_sum(p_ref):
    s = jnp.concatenate([p_ref[0], p_ref[1], p_ref[2], p_ref[3]], axis=1)
    cnt = jnp.clip(s[:, :1], 1.0, None)
    return jax.nn.relu(s / cnt)


def _mid_body(p_ref, w_ref, o_ref):
    z = _z_from_sum(p_ref)
    mm = jnp.dot(z, w_ref[...], preferred_element_type=jnp.float32)
    col = lax.broadcasted_iota(jnp.int32, mm.shape, 1)
    _split_quarters(jnp.where(col == 0, 1.0, mm), o_ref)


def _mid(p, wt):
    return pl.pallas_call(
        _mid_body,
        grid=(N_NODES // BLK,),
        in_specs=[pl.BlockSpec((4, BLK, DQ), lambda i: (0, i, 0)),
                  pl.BlockSpec((D, D), lambda i: (0, 0))],
        out_specs=pl.BlockSpec((4, BLK, DQ), lambda i: (0, i, 0)),
        out_shape=jax.ShapeDtypeStruct((4, N_NODES, DQ), jnp.float32),
    )(p, wt)


def _head_body(p_ref, b_ref, w_ref, bias_ref, o_ref, accp, accc):
    i = pl.program_id(0)

    @pl.when(i == 0)
    def _():
        accp[...] = jnp.zeros_like(accp)
        accc[...] = jnp.zeros_like(accc)

    z = _z_from_sum(p_ref)
    b = b_ref[0, 0, :]
    onehot = (b[:, None] == lax.broadcasted_iota(jnp.int32, (BLK, N_GRAPHS), 1)
              ).astype(jnp.float32)
    accp[...] += lax.dot_general(onehot, z, (((0,), (0,)), ((), ())),
                                 preferred_element_type=jnp.float32)
    accc[...] += jnp.sum(onehot, axis=0)[:, None]

    @pl.when(i == pl.num_programs(0) - 1)
    def _():
        pooled = accp[...] / jnp.clip(accc[...], 1.0, None)
        o_ref[...] = (jnp.dot(pooled, w_ref[...],
                              preferred_element_type=jnp.float32)
                      + bias_ref[...])


def _head(p, batch3d, wt, bias):
    return pl.pallas_call(
        _head_body,
        grid=(N_NODES // BLK,),
        in_specs=[pl.BlockSpec((4, BLK, DQ), lambda i: (0, i, 0)),
                  pl.BlockSpec((1, 1, BLK), lambda i: (i, 0, 0)),
                  pl.BlockSpec((D, D), lambda i: (0, 0)),
                  pl.BlockSpec((1, D), lambda i: (0, 0))],
        out_specs=pl.BlockSpec((N_GRAPHS, D), lambda i: (0, 0)),
        out_shape=jax.ShapeDtypeStruct((N_GRAPHS, D), jnp.float32),
        scratch_shapes=[pltpu.VMEM((N_GRAPHS, D), jnp.float32),
                        pltpu.VMEM((N_GRAPHS, 1), jnp.float32)],
    )(p, batch3d, wt, bias)


def kernel(x, edge_index, batch, W1, W2, W4, b4):
    f32 = jnp.float32
    w1t = W1.at[:, 0].set(0.0).T
    w2t = W2.at[:, 0].set(0.0).T
    w4t = W4.at[:, 0].set(0.0).T
    bias = b4.reshape(1, D)

    n_e = edge_index.shape[1]
    pad = E_PAD - n_e
    srcs = jnp.concatenate(
        [edge_index[0], jnp.zeros((pad,), jnp.int32)]).reshape(
            NS, N_CHUNKS, CHUNK)
    dsts = jnp.concatenate(
        [edge_index[1], jnp.full((pad,), DST_PAD, jnp.int32)]).reshape(
            NS, N_CHUNKS, CHUNK)
    zeros = jnp.zeros((N_PAD, DQ), f32)
    batch3d = batch.reshape(N_NODES // BLK, 1, BLK)

    xt1 = _mm_ones(x, w1t)
    p1 = _agg(xt1, srcs, dsts, zeros)
    xt2 = _mid(p1, w2t)
    p2 = _agg(xt2, srcs, dsts, zeros)
    return _head(p2, batch3d, w4t, bias)


# speedup vs baseline: 1.0087x; 1.0087x over previous
import functools

import jax
import jax.numpy as jnp
from jax import lax
from jax.experimental import pallas as pl
from jax.experimental.pallas import tpu as pltpu
from jax.experimental.pallas import tpu_sc as plsc

N_NODES = 10000
N_GRAPHS = 64
D = 128
DQ = 32

NC, NS = 2, 16
CHUNK = 128
N_CHUNKS = 160
NBUF = 4
E_PAD = NS * N_CHUNKS * CHUNK
N_PAD = 10240
ROWS_PER_TILE = N_PAD // NS
DST_PAD = N_NODES + 16

BLK = 2000


def _agg_body(table, srcs, dsts, zeros, out, idx_s, idx_d, rows, acc,
              s0, s1, s2, s3):
    gsems = (s0, s1, s2, s3)
    cid = lax.axis_index("c")
    sid = lax.axis_index("s")
    pltpu.sync_copy(srcs.at[sid], idx_s)
    pltpu.sync_copy(dsts.at[sid], idx_d)
    r0 = sid * ROWS_PER_TILE
    for q in range(2):
        pltpu.sync_copy(zeros.at[pl.ds(r0, ROWS_PER_TILE)],
                        acc.at[q, pl.ds(r0, ROWS_PER_TILE)])
    plsc.subcore_barrier()
    qs = [table.at[2 * cid], table.at[2 * cid + 1]]
    accq = [acc.at[0], acc.at[1]]
    for b in range(NBUF):
        for q in range(2):
            pltpu.async_copy(qs[q].at[idx_s.at[b]], rows.at[b, q], gsems[b])

    def group(g, carry):
        j0 = g * NBUF
        for b in range(NBUF):
            j = j0 + b
            for q in range(2):
                pltpu.make_async_copy(qs[q].at[idx_s.at[j]], rows.at[b, q],
                                      gsems[b]).wait()
            for q in range(2):
                pltpu.sync_copy(rows.at[b, q], accq[q].at[idx_d.at[j]],
                                add=True)
            nxt = j + NBUF

            @pl.when(nxt < N_CHUNKS)
            def _():
                for q in range(2):
                    pltpu.async_copy(qs[q].at[idx_s.at[nxt]], rows.at[b, q],
                                     gsems[b])
        return carry

    lax.fori_loop(0, N_CHUNKS // NBUF, group, 0)
    plsc.subcore_barrier()
    for q in range(2):
        pltpu.sync_copy(acc.at[q, pl.ds(r0, ROWS_PER_TILE)],
                        out.at[2 * cid + q, pl.ds(r0, ROWS_PER_TILE)])


@functools.cache
def _make_agg():
    return pl.kernel(
        _agg_body,
        out_type=jax.ShapeDtypeStruct((4, N_PAD, DQ), jnp.float32),
        mesh=plsc.VectorSubcoreMesh(core_axis_name="c", subcore_axis_name="s"),
        scratch_types=[
            pltpu.VMEM((N_CHUNKS, CHUNK), jnp.int32),
            pltpu.VMEM((N_CHUNKS, CHUNK), jnp.int32),
            pltpu.VMEM((NBUF, 2, CHUNK, DQ), jnp.float32),
            pltpu.VMEM_SHARED((2, N_PAD, DQ), jnp.float32),
            pltpu.SemaphoreType.DMA,
            pltpu.SemaphoreType.DMA,
            pltpu.SemaphoreType.DMA,
            pltpu.SemaphoreType.DMA,
        ],
        compiler_params=pltpu.CompilerParams(use_tc_tiling_on_sc=False),
    )


def _agg(table, srcs, dsts, zeros):
    return _make_agg()(table, srcs, dsts, zeros)


def _split_quarters(mm, o_ref):
    for q in range(4):
        o_ref[q, :, :] = mm[:, DQ * q:DQ * (q + 1)]


def _mm_ones_body(x_ref, w_ref, o_ref):
    xb = x_ref[...]
    x0 = xb[:, :1]
    sq = jnp.sum(xb * xb, axis=1, keepdims=True) - x0 * x0
    ynorm = jnp.clip(jnp.sqrt(jnp.clip(sq, 0.0, None)), 1e-15, None)
    theta = jnp.clip(x0, 1.0 + 1e-7, None)
    ar = jnp.log(theta + jnp.sqrt(jnp.clip(theta * theta - 1.0, 1e-15, None)))
    mm = jnp.dot(xb * (ar / ynorm), w_ref[...],
                 preferred_element_type=jnp.float32)
    col = lax.broadcasted_iota(jnp.int32, mm.shape, 1)
    _split_quarters(jnp.where(col == 0, 1.0, mm), o_ref)


def _mm_ones(xp, wt):
    return pl.pallas_call(
        _mm_ones_body,
        grid=(N_NODES // BLK,),
        in_specs=[pl.BlockSpec((BLK, D), lambda i: (i, 0)),
                  pl.BlockSpec((D, D), lambda i: (0, 0))],
        out_specs=pl.BlockSpec((4, BLK, DQ), lambda i: (0, i, 0)),
        out_shape=jax.ShapeDtypeStruct((4, N_NODES, DQ), jnp.float32),
    )(xp, wt)


def _z_from_sum(p_ref):
    s = jnp.concatenate([p_ref[0], p_ref[1], p_ref[2], p_ref[3]], axis=1)
    cnt = jnp.clip(s[:, :1], 1.0, None)
    return jax.nn.relu(s / cnt)


def _mid_body(p_ref, w_ref, o_ref):
    z = _z_from_sum(p_ref)
    mm = jnp.dot(z, w_ref[...], preferred_element_type=jnp.float32)
    col = lax.broadcasted_iota(jnp.int32, mm.shape, 1)
    _split_quarters(jnp.where(col == 0, 1.0, mm), o_ref)


def _mid(p, wt):
    return pl.pallas_call(
        _mid_body,
        grid=(N_NODES // BLK,),
        in_specs=[pl.BlockSpec((4, BLK, DQ), lambda i: (0, i, 0)),
                  pl.BlockSpec((D, D), lambda i: (0, 0))],
        out_specs=pl.BlockSpec((4, BLK, DQ), lambda i: (0, i, 0)),
        out_shape=jax.ShapeDtypeStruct((4, N_NODES, DQ), jnp.float32),
    )(p, wt)


def _head_body(p_ref, b_ref, w_ref, bias_ref, o_ref, accp, accc):
    i = pl.program_id(0)

    @pl.when(i == 0)
    def _():
        accp[...] = jnp.zeros_like(accp)
        accc[...] = jnp.zeros_like(accc)

    z = _z_from_sum(p_ref)
    b = b_ref[0, 0, :]
    onehot = (b[:, None] == lax.broadcasted_iota(jnp.int32, (BLK, N_GRAPHS), 1)
              ).astype(jnp.float32)
    accp[...] += lax.dot_general(onehot, z, (((0,), (0,)), ((), ())),
                                 preferred_element_type=jnp.float32)
    accc[...] += jnp.sum(onehot, axis=0)[:, None]

    @pl.when(i == pl.num_programs(0) - 1)
    def _():
        pooled = accp[...] / jnp.clip(accc[...], 1.0, None)
        o_ref[...] = (jnp.dot(pooled, w_ref[...],
                              preferred_element_type=jnp.float32)
                      + bias_ref[...])


def _head(p, batch3d, wt, bias):
    return pl.pallas_call(
        _head_body,
        grid=(N_NODES // BLK,),
        in_specs=[pl.BlockSpec((4, BLK, DQ), lambda i: (0, i, 0)),
                  pl.BlockSpec((1, 1, BLK), lambda i: (i, 0, 0)),
                  pl.BlockSpec((D, D), lambda i: (0, 0)),
                  pl.BlockSpec((1, D), lambda i: (0, 0))],
        out_specs=pl.BlockSpec((N_GRAPHS, D), lambda i: (0, 0)),
        out_shape=jax.ShapeDtypeStruct((N_GRAPHS, D), jnp.float32),
        scratch_shapes=[pltpu.VMEM((N_GRAPHS, D), jnp.float32),
                        pltpu.VMEM((N_GRAPHS, 1), jnp.float32)],
    )(p, batch3d, wt, bias)


def kernel(x, edge_index, batch, W1, W2, W4, b4):
    f32 = jnp.float32
    w1t = W1.at[:, 0].set(0.0).T
    w2t = W2.at[:, 0].set(0.0).T
    w4t = W4.at[:, 0].set(0.0).T
    bias = b4.reshape(1, D)

    n_e = edge_index.shape[1]
    pad = E_PAD - n_e
    srcs = jnp.concatenate(
        [edge_index[0], jnp.zeros((pad,), jnp.int32)]).reshape(
            NS, N_CHUNKS, CHUNK)
    dsts = jnp.concatenate(
        [edge_index[1], jnp.full((pad,), DST_PAD, jnp.int32)]).reshape(
            NS, N_CHUNKS, CHUNK)
    zeros = jnp.zeros((N_PAD, DQ), f32)
    batch3d = batch.reshape(N_NODES // BLK, 1, BLK)

    xt1 = _mm_ones(x, w1t)
    p1 = _agg(xt1, srcs, dsts, zeros)
    xt2 = _mid(p1, w2t)
    p2 = _agg(xt2, srcs, dsts, zeros)
    return _head(p2, batch3d, w4t, bias)
